# baseline (speedup 1.0000x reference)
import jax
import jax.numpy as jnp
from jax import lax
from jax.experimental import pallas as pl
from jax.experimental.pallas import tpu as pltpu

_B, _N, _Q = 16, 2048, 128
_CHUNK = 128
_K = _N // _CHUNK


def _fused_body(xs_ref, ys_ref, s_ref, w_ref, b_ref, out_ref):
    sc = s_ref[...]
    cols = []
    for b in range(_B):
        acc = None
        for k in range(_K):
            xc = lax.slice(xs_ref[...], (b, k * _CHUNK), (b + 1, (k + 1) * _CHUNK))
            yc = lax.slice(ys_ref[...], (b, k * _CHUNK), (b + 1, (k + 1) * _CHUNK))
            phi = jnp.maximum(jnp.minimum(sc - xc, yc - sc), 0.0)
            acc = phi if acc is None else acc + phi
        cols.append(jnp.sum(acc, axis=1, keepdims=True))
    p = jnp.concatenate(cols, axis=1)
    r = lax.dot_general(
        w_ref[...], p, (((1,), (0,)), ((), ())),
        preferred_element_type=jnp.float32,
    )
    out_ref[...] = jnp.maximum(r + b_ref[...], 0.0)


_fused = pl.pallas_call(
    _fused_body,
    out_shape=jax.ShapeDtypeStruct((_Q, _B), jnp.float32),
)


def kernel(diagram, samples, rho_w, rho_b):
    d2 = diagram.reshape(_B, 2 * _N)
    xs = jax.lax.slice(d2, (0, 0), (_B, _N))
    ys = jax.lax.slice(d2, (0, _N), (_B, 2 * _N))
    out_t = _fused(xs, ys, samples.reshape(_Q, 1), rho_w,
                   rho_b.reshape(_Q, 1))
    return out_t


# speedup vs baseline: 1.1130x; 1.1130x over previous
import jax
import jax.numpy as jnp
from jax import lax
from jax.experimental import pallas as pl
from jax.experimental.pallas import tpu as pltpu

_B, _N, _Q = 16, 2048, 128
_CHUNK = 128
_K = _N // _CHUNK


def _fused_body(xs_ref, ys_ref, s_ref, w_ref, b_ref, out_ref):
    sc = s_ref[...]
    cols = []
    for b in range(_B):
        acc = None
        for k in range(_K):
            xc = lax.slice(xs_ref[...], (b, k * _CHUNK), (b + 1, (k + 1) * _CHUNK))
            yc = lax.slice(ys_ref[...], (b, k * _CHUNK), (b + 1, (k + 1) * _CHUNK))
            phi = jnp.maximum(jnp.minimum(sc - xc, yc - sc), 0.0)
            acc = phi if acc is None else acc + phi
        cols.append(jnp.sum(acc, axis=1, keepdims=True))
    p = jnp.concatenate(cols, axis=1)
    r = lax.dot_general(
        w_ref[...], p, (((1,), (0,)), ((), ())),
        preferred_element_type=jnp.float32,
    )
    out_ref[...] = jnp.maximum(r + b_ref[...], 0.0)


_fused = pl.pallas_call(
    _fused_body,
    out_shape=jax.ShapeDtypeStruct((_Q, _B), jnp.float32),
)


def kernel(diagram, samples, rho_w, rho_b):
    xs = diagram[:, :, 0]
    ys = diagram[:, :, 1]
    out_t = _fused(xs, ys, samples.reshape(_Q, 1), rho_w,
                   rho_b.reshape(_Q, 1))
    return out_t.T


# speedup vs baseline: 1.2053x; 1.0829x over previous
import jax
import jax.numpy as jnp
from jax import lax
from jax.experimental import pallas as pl
from jax.experimental.pallas import tpu as pltpu

_B, _N, _Q = 16, 2048, 128
_CHUNK = 128
_K = _N // _CHUNK


def _fused_body(xy_ref, s_ref, w_ref, b_ref, out_ref):
    sc = s_ref[...]
    v = xy_ref[...]
    cols = []
    for b in range(_B):
        acc = None
        for k in range(_K):
            xc = lax.slice(v, (32 * b + k, 0), (32 * b + k + 1, _CHUNK))
            yc = lax.slice(v, (32 * b + 16 + k, 0), (32 * b + 17 + k, _CHUNK))
            phi = jnp.maximum(jnp.minimum(sc - xc, yc - sc), 0.0)
            acc = phi if acc is None else acc + phi
        cols.append(jnp.sum(acc, axis=1, keepdims=True))
    p = jnp.concatenate(cols, axis=1)
    r = lax.dot_general(
        w_ref[...], p, (((1,), (0,)), ((), ())),
        preferred_element_type=jnp.float32,
    )
    out_ref[...] = jnp.maximum(r + b_ref[...], 0.0)


_fused = pl.pallas_call(
    _fused_body,
    out_shape=jax.ShapeDtypeStruct((_Q, _B), jnp.float32),
)


def kernel(diagram, samples, rho_w, rho_b):
    xy = diagram.transpose(0, 2, 1).reshape(32 * _B, _CHUNK)
    out_t = _fused(xy, samples.reshape(_Q, 1), rho_w, rho_b.reshape(_Q, 1))
    return out_t.T


# speedup vs baseline: 1.4758x; 1.2244x over previous
import jax
import jax.numpy as jnp
from jax import lax
from jax.experimental import pallas as pl
from jax.experimental.pallas import tpu as pltpu

_B, _N, _Q = 16, 2048, 128
_CHUNK = 128
_K = _N // _CHUNK
_GB = 4
_STEPS = _B // _GB


def _fused_body(xy_ref, s_ref, w_ref, b_ref, out_ref, p_ref):
    g = pl.program_id(0)
    sc = s_ref[...]
    v = xy_ref[...]
    for j in range(_GB):
        acc = None
        for k in range(_K):
            xc = lax.slice(v, (32 * j + k, 0), (32 * j + k + 1, _CHUNK))
            yc = lax.slice(v, (32 * j + 16 + k, 0), (32 * j + 17 + k, _CHUNK))
            phi = jnp.maximum(jnp.minimum(sc - xc, yc - sc), 0.0)
            acc = phi if acc is None else acc + phi
        pooled = jnp.sum(acc, axis=1, keepdims=True)
        p_ref[pl.ds(g * _GB + j, 1), :] = pooled.reshape(1, _Q)

    @pl.when(g == _STEPS - 1)
    def _():
        r = lax.dot_general(
            p_ref[...], w_ref[...], (((1,), (1,)), ((), ())),
            preferred_element_type=jnp.float32,
        )
        out_ref[...] = jnp.maximum(r + b_ref[...], 0.0)


_fused = pl.pallas_call(
    _fused_body,
    grid=(_STEPS,),
    in_specs=[
        pl.BlockSpec((32 * _GB, _CHUNK), lambda g: (g, 0)),
        pl.BlockSpec((_Q, 1), lambda g: (0, 0)),
        pl.BlockSpec((_Q, _Q), lambda g: (0, 0)),
        pl.BlockSpec((1, _Q), lambda g: (0, 0)),
    ],
    out_specs=pl.BlockSpec((_B, _Q), lambda g: (0, 0)),
    out_shape=jax.ShapeDtypeStruct((_B, _Q), jnp.float32),
    scratch_shapes=[pltpu.VMEM((_B, _Q), jnp.float32)],
)


def kernel(diagram, samples, rho_w, rho_b):
    xy = diagram.transpose(0, 2, 1).reshape(32 * _B, _CHUNK)
    return _fused(xy, samples.reshape(_Q, 1), rho_w, rho_b.reshape(1, _Q))
